# baseline (speedup 1.0000x reference)
import jax
import jax.numpy as jnp
from jax import lax
from jax.experimental import pallas as pl
from jax.experimental.pallas import tpu as pltpu
from jax.experimental.pallas import tpu_sc as plsc

B = 64
C = 3
H = 512
W = 512
ROW = H * W
NBLK = H
BLK = W
TOPK = 8
NWORKERS = 32
ROWS_PER_W = B // NWORKERS
_BIG = 1 << 20


def _tc_max_body(pred_ref, target_ref, mx_ref):
    for rr in range(2):
        p = pred_ref[rr]
        t = target_ref[rr]
        d = t - p
        d = d * d
        s = d[0] + d[1] + d[2]
        mx_ref[rr, 0] = jnp.max(s, axis=1)


def _tc_max(pred, target, off):
    ob = off // 2
    return pl.pallas_call(
        _tc_max_body,
        grid=(B // 4,),
        in_specs=[
            pl.BlockSpec((2, C, H, W), lambda b: (b + ob, 0, 0, 0)),
            pl.BlockSpec((2, C, H, W), lambda b: (b + ob, 0, 0, 0)),
        ],
        out_specs=pl.BlockSpec((2, 1, NBLK), lambda b: (b, 0, 0)),
        out_shape=jax.ShapeDtypeStruct((B // 2, 1, NBLK), jnp.float32),
    )(pred, target)


def _insert8(ms, t):
    for k in range(TOPK):
        hi = jnp.maximum(ms[k], t)
        t = jnp.minimum(ms[k], t)
        ms[k] = hi
    return ms


def _merge_top8(ms):
    total = jnp.zeros((16,), jnp.float32)
    remaining = jnp.int32(TOPK)
    for _ in range(TOPK):
        m = ms[0]
        for k in range(1, TOPK):
            m = jnp.maximum(m, ms[k])
        mx = jnp.max(m)
        cnt = jnp.zeros((16,), jnp.int32)
        for k in range(TOPK):
            cnt = cnt + jnp.where(ms[k] == mx, jnp.int32(1), jnp.int32(0))
        c = jnp.sum(cnt)
        take = jnp.minimum(c, remaining)
        total = total + jnp.broadcast_to(mx * take.astype(jnp.float32), (16,))
        for k in range(TOPK):
            ms[k] = jnp.where(ms[k] == mx, jnp.float32(-1.0), ms[k])
        remaining = remaining - take
    return total


def _sc_select(row, off, slot, mx_hbm, pred_hbm, target_hbm, mbuf, pb, tb, sem):
    pltpu.sync_copy(mx_hbm.at[row, 0], mbuf)
    grow = row + off
    nv = NBLK // 16
    iota = lax.iota(jnp.int32, 16)
    copies = []
    for it in range(TOPK):
        vs = [mbuf[pl.ds(j * 16, 16)] for j in range(nv)]
        m = vs[0]
        for j in range(1, nv):
            m = jnp.maximum(m, vs[j])
        mx = jnp.max(m)
        cand = jnp.where(vs[0] == mx, iota, _BIG)
        for j in range(1, nv):
            cand = jnp.minimum(cand, jnp.where(vs[j] == mx, iota + j * 16, _BIG))
        istar = jnp.min(cand)
        for ch in range(C):
            copies.append(pltpu.async_copy(
                pred_hbm.at[grow, ch, pl.ds(istar, 1)],
                pb.at[slot, it, ch], sem))
            copies.append(pltpu.async_copy(
                target_hbm.at[grow, ch, pl.ds(istar, 1)],
                tb.at[slot, it, ch], sem))
        g = lax.shift_right_logical(istar, 4)
        lane = istar - g * 16
        vg = mbuf[pl.ds(g * 16, 16)]
        mbuf[pl.ds(g * 16, 16)] = jnp.where(iota == lane,
                                            jnp.float32(-1.0), vg)
    return copies


def _sc_rescan(row, slot, out_hbm, pb, tb, obuf, copies):
    for cp in copies:
        cp.wait()

    def body(i, carry):
        ms = list(carry)
        for u in range(8):
            idx = i * 8 + u
            it = lax.shift_right_logical(idx, 5)
            col = (idx - it * 32) * 16
            acc = None
            for ch in range(C):
                x = (tb[slot, it, ch, 0, pl.ds(col, 16)]
                     - pb[slot, it, ch, 0, pl.ds(col, 16)])
                sq = x * x
                acc = sq if acc is None else acc + sq
            ms = _insert8(ms, acc)
        return tuple(ms)

    init = tuple(jnp.full((16,), -1.0, jnp.float32) for _ in range(TOPK))
    ms = list(lax.fori_loop(0, TOPK * BLK // 128, body, init))

    obuf[...] = _merge_top8(ms)
    pltpu.sync_copy(obuf, out_hbm.at[row])


def _make_sc_topk_body(off):
    def _sc_topk_body(mx_hbm, pred_hbm, target_hbm, out_hbm,
                      mbuf, pb, tb, obuf, sem0, sem1):
        wid = lax.axis_index("s") * 2 + lax.axis_index("c")
        cp0 = _sc_select(wid, off, 0, mx_hbm, pred_hbm, target_hbm,
                         mbuf, pb, tb, sem0)
        _sc_rescan(wid, 0, out_hbm, pb, tb, obuf, cp0)
    return _sc_topk_body


def _sc_topk(mx, pred, target, off):
    fn = pl.kernel(
        _make_sc_topk_body(off),
        out_type=jax.ShapeDtypeStruct((B // 2, 16), jnp.float32),
        mesh=plsc.VectorSubcoreMesh(
            core_axis_name="c", subcore_axis_name="s",
            num_cores=2, num_subcores=16),
        scratch_types=[
            pltpu.VMEM((NBLK,), jnp.float32),
            pltpu.VMEM((2, TOPK, C, 1, BLK), jnp.float32),
            pltpu.VMEM((2, TOPK, C, 1, BLK), jnp.float32),
            pltpu.VMEM((16,), jnp.float32),
            pltpu.SemaphoreType.DMA,
            pltpu.SemaphoreType.DMA,
        ],
        compiler_params=pltpu.CompilerParams(needs_layout_passes=False),
    )
    return fn(mx, pred, target)


def _tc_mean_body(a_ref, b_ref, out_ref):
    s = (jnp.sum(a_ref[...][:, 0:1], keepdims=True)
         + jnp.sum(b_ref[...][:, 0:1], keepdims=True))
    out_ref[...] = s / jnp.float32(B * TOPK)


def _tc_mean(sums_a, sums_b):
    return pl.pallas_call(
        _tc_mean_body,
        out_shape=jax.ShapeDtypeStruct((1, 1), jnp.float32),
    )(sums_a, sums_b)


def kernel(pred, target):
    mx1 = _tc_max(pred, target, 0)
    sums1 = _sc_topk(mx1, pred, target, 0)
    mx2 = _tc_max(pred, target, B // 2)
    sums2 = _sc_topk(mx2, pred, target, B // 2)
    return _tc_mean(sums1, sums2)[0, 0]


# speedup vs baseline: 1.0027x; 1.0027x over previous
import jax
import jax.numpy as jnp
from jax import lax
from jax.experimental import pallas as pl
from jax.experimental.pallas import tpu as pltpu
from jax.experimental.pallas import tpu_sc as plsc

B = 64
C = 3
H = 512
W = 512
ROW = H * W
NBLK = H
BLK = W
TOPK = 8
NWORKERS = 32
ROWS_PER_W = B // NWORKERS
_BIG = 1 << 20


def _tc_max_body(pred_ref, target_ref, mx_ref):
    for rr in range(2):
        p = pred_ref[rr]
        t = target_ref[rr]
        d = t - p
        d = d * d
        s = d[0] + d[1] + d[2]
        mx_ref[rr, 0] = jnp.max(s, axis=1)


def _tc_max(pred, target):
    return pl.pallas_call(
        _tc_max_body,
        grid=(B // 2,),
        in_specs=[
            pl.BlockSpec((2, C, H, W), lambda b: (b, 0, 0, 0)),
            pl.BlockSpec((2, C, H, W), lambda b: (b, 0, 0, 0)),
        ],
        out_specs=pl.BlockSpec((2, 1, NBLK), lambda b: (b, 0, 0)),
        out_shape=jax.ShapeDtypeStruct((B, 1, NBLK), jnp.float32),
    )(pred, target)


def _insert8(ms, t):
    for k in range(TOPK):
        hi = jnp.maximum(ms[k], t)
        t = jnp.minimum(ms[k], t)
        ms[k] = hi
    return ms


def _merge_top8(ms):
    total = jnp.zeros((16,), jnp.float32)
    remaining = jnp.int32(TOPK)
    for _ in range(TOPK):
        m = ms[0]
        for k in range(1, TOPK):
            m = jnp.maximum(m, ms[k])
        mx = jnp.max(m)
        cnt = jnp.zeros((16,), jnp.int32)
        for k in range(TOPK):
            cnt = cnt + jnp.where(ms[k] == mx, jnp.int32(1), jnp.int32(0))
        c = jnp.sum(cnt)
        take = jnp.minimum(c, remaining)
        total = total + jnp.broadcast_to(mx * take.astype(jnp.float32), (16,))
        for k in range(TOPK):
            ms[k] = jnp.where(ms[k] == mx, jnp.float32(-1.0), ms[k])
        remaining = remaining - take
    return total


def _sc_select(row, slot, mx_hbm, pred_hbm, target_hbm, mbuf, pb, tb, sem):
    pltpu.sync_copy(mx_hbm.at[row, 0], mbuf)
    nv = NBLK // 16
    iota = lax.iota(jnp.int32, 16)
    copies = []
    for it in range(TOPK):
        vs = [mbuf[pl.ds(j * 16, 16)] for j in range(nv)]
        m = vs[0]
        for j in range(1, nv):
            m = jnp.maximum(m, vs[j])
        mx = jnp.max(m)
        cand = jnp.where(vs[0] == mx, iota, _BIG)
        for j in range(1, nv):
            cand = jnp.minimum(cand, jnp.where(vs[j] == mx, iota + j * 16, _BIG))
        istar = jnp.min(cand)
        for ch in range(C):
            copies.append(pltpu.async_copy(
                pred_hbm.at[row, ch, pl.ds(istar, 1)],
                pb.at[slot, it, ch], sem))
            copies.append(pltpu.async_copy(
                target_hbm.at[row, ch, pl.ds(istar, 1)],
                tb.at[slot, it, ch], sem))
        g = lax.shift_right_logical(istar, 4)
        lane = istar - g * 16
        vg = mbuf[pl.ds(g * 16, 16)]
        mbuf[pl.ds(g * 16, 16)] = jnp.where(iota == lane,
                                            jnp.float32(-1.0), vg)
    return copies


def _sc_rescan(row, slot, out_hbm, pb, tb, obuf, copies):
    for cp in copies:
        cp.wait()

    def body(i, carry):
        ms = list(carry)
        for u in range(8):
            idx = i * 8 + u
            it = lax.shift_right_logical(idx, 5)
            col = (idx - it * 32) * 16
            acc = None
            for ch in range(C):
                x = (tb[slot, it, ch, 0, pl.ds(col, 16)]
                     - pb[slot, it, ch, 0, pl.ds(col, 16)])
                sq = x * x
                acc = sq if acc is None else acc + sq
            ms = _insert8(ms, acc)
        return tuple(ms)

    init = tuple(jnp.full((16,), -1.0, jnp.float32) for _ in range(TOPK))
    ms = list(lax.fori_loop(0, TOPK * BLK // 128, body, init))

    obuf[...] = _merge_top8(ms)
    pltpu.sync_copy(obuf, out_hbm.at[row])


def _sc_topk_body(mx_hbm, pred_hbm, target_hbm, out_hbm,
                  mbuf, pb, tb, obuf, sem0, sem1):
    wid = lax.axis_index("s") * 2 + lax.axis_index("c")
    row0 = wid * ROWS_PER_W
    row1 = row0 + 1
    cp0 = _sc_select(row0, 0, mx_hbm, pred_hbm, target_hbm, mbuf, pb, tb, sem0)
    cp1 = _sc_select(row1, 1, mx_hbm, pred_hbm, target_hbm, mbuf, pb, tb, sem1)
    _sc_rescan(row0, 0, out_hbm, pb, tb, obuf, cp0)
    _sc_rescan(row1, 1, out_hbm, pb, tb, obuf, cp1)


def _sc_topk(mx, pred, target):
    fn = pl.kernel(
        _sc_topk_body,
        out_type=jax.ShapeDtypeStruct((B, 16), jnp.float32),
        mesh=plsc.VectorSubcoreMesh(
            core_axis_name="c", subcore_axis_name="s",
            num_cores=2, num_subcores=16),
        scratch_types=[
            pltpu.VMEM((NBLK,), jnp.float32),
            pltpu.VMEM((2, TOPK, C, 1, BLK), jnp.float32),
            pltpu.VMEM((2, TOPK, C, 1, BLK), jnp.float32),
            pltpu.VMEM((16,), jnp.float32),
            pltpu.SemaphoreType.DMA,
            pltpu.SemaphoreType.DMA,
        ],
        compiler_params=pltpu.CompilerParams(needs_layout_passes=False),
    )
    return fn(mx, pred, target)


def _tc_mean_body(sums_ref, out_ref):
    s = jnp.sum(sums_ref[...][:, 0:1], keepdims=True)
    out_ref[...] = s / jnp.float32(B * TOPK)


def _tc_mean(sums):
    return pl.pallas_call(
        _tc_mean_body,
        out_shape=jax.ShapeDtypeStruct((1, 1), jnp.float32),
    )(sums)


def kernel(pred, target):
    mx = _tc_max(pred, target)
    sums = _sc_topk(mx, pred, target)
    return _tc_mean(sums)[0, 0]
